# baseline (speedup 1.0000x reference)
import functools

import jax
import jax.numpy as jnp
from jax import lax
from jax.experimental import pallas as pl
from jax.experimental.pallas import tpu as pltpu
from jax.experimental.pallas import tpu_sc as plsc

_NSC = 2
_NSUB = 16
_NW = _NSC * _NSUB
_CAP = 4096


def _srl(x, n):
    return lax.shift_right_logical(x, jnp.int32(n))


def _sll(x, n):
    return lax.shift_left(x, jnp.int32(n))


def _fp16_bits_to_f32(u):
    s = _srl(u, 15) & 1
    e = _srl(u, 10) & 31
    man = u & 1023
    bits = _sll(s, 31) | _sll(e + 112, 23) | _sll(man, 13)
    val_n = lax.bitcast_convert_type(bits, jnp.float32)
    sgn = 1.0 - 2.0 * s.astype(jnp.float32)
    val_s = sgn * man.astype(jnp.float32) * jnp.float32(2.0 ** -24)
    return jnp.where(e == 0, val_s, val_n)


def _stats_kernel(lo_ref, hi_ref, scale_ref, zero_ref):
    lo = lo_ref[...]
    hi = hi_ref[...]
    c0 = _srl(hi, 22) & 255

    def so(j):
        u = c0[2 * j : 2 * j + 1, :] | _sll(c0[2 * j + 1 : 2 * j + 2, :], 8)
        return _fp16_bits_to_f32(u)

    ss, sz, zs, zz = so(0), so(1), so(2), so(3)
    ws = (lo & 7).astype(jnp.float32)
    wz = (_srl(lo, 3) & 7).astype(jnp.float32)
    scale_ref[...] = ws * ss + sz
    zero_ref[...] = wz * zs + zz


def _mm_kernel(x_ref, lo_ref, hi_ref, sc_ref, zr_ref, dwt_ref, out_ref):
    lo = lo_ref[...]
    hi = hi_ref[...]
    planes = []
    for i in range(16):
        s = 6 + 3 * i
        if s + 3 <= 32:
            p = _srl(lo, s) & 7
        elif s < 32:
            p = (_srl(lo, 30) & 3) | _sll(hi & 1, 2)
        else:
            p = _srl(hi, s - 32) & 7
        planes.append(p)
    wq = jnp.stack(planes, axis=0).astype(jnp.float32)
    sc = sc_ref[...]
    zr = zr_ref[...]
    w = sc[None, :, :] * (wq - zr[None, :, :])
    _, btn, bm = w.shape
    w = w.reshape(btn * 16, bm) + dwt_ref[...]
    acc = jnp.dot(
        x_ref[...], w.astype(jnp.bfloat16), preferred_element_type=jnp.float32
    )
    k = pl.program_id(1)

    @pl.when(k == 0)
    def _():
        out_ref[...] = acc

    @pl.when(k > 0)
    def _():
        out_ref[...] += acc


def _sc_scatter_body(offs_hbm, cv_hbm, dw_hbm, offs_v, buf_v, stage_v):
    mn = dw_hbm.shape[0]
    n = buf_v.shape[0] // 17
    m = mn // n
    rows_per_w = m // _NW
    n_groups = rows_per_w // 16
    wid = lax.axis_index("s") * jnp.int32(_NSC) + lax.axis_index("c")
    base_row = wid * jnp.int32(rows_per_w)
    pltpu.sync_copy(offs_hbm.at[pl.ds(pl.multiple_of(base_row, 8), 144)], offs_v)

    def full(v):
        return jnp.full((16,), v, jnp.int32)

    z16f = jnp.zeros((16,), jnp.float32)
    iota = lax.iota(jnp.int32, 16)
    c_n = full(n)
    c_trash = full(16)
    c_lo16 = full(0xFFFF)
    one = full(1)
    zero = full(0)

    def decode_val(u):
        s = lax.shift_right_logical(u, full(15)) & one
        ex = lax.shift_right_logical(u, full(10)) & full(31)
        man = u & full(1023)
        bits = (
            lax.shift_left(s, full(31))
            | lax.shift_left(ex + full(112), full(23))
            | lax.shift_left(man, full(13))
        )
        val_n = lax.bitcast_convert_type(bits, jnp.float32)
        sgn = jnp.full((16,), 1.0, jnp.float32) - jnp.full(
            (16,), 2.0, jnp.float32
        ) * s.astype(jnp.float32)
        val_s = (
            sgn
            * man.astype(jnp.float32)
            * jnp.full((16,), 2.0 ** -24, jnp.float32)
        )
        return jnp.where(ex == zero, val_s, val_n)

    def zero_body(i, carry):
        buf_v[pl.ds(i * jnp.int32(16), 16)] = z16f
        return carry

    lax.fori_loop(jnp.int32(0), jnp.int32(17 * n // 16), zero_body, 0)

    def entry_sweep(start, end, ojs, accumulate):
        p0 = start & jnp.int32(-8)
        n_pieces = lax.div(end - p0 + jnp.int32(_CAP - 1), jnp.int32(_CAP))

        def piece_body(pi, carry):
            pstart = p0 + pi * jnp.int32(_CAP)
            pltpu.sync_copy(
                cv_hbm.at[pl.ds(pl.multiple_of(pstart, 8), _CAP)], stage_v
            )
            nsub = lax.div(
                jnp.minimum(end - pstart, jnp.int32(_CAP)) + jnp.int32(15),
                jnp.int32(16),
            )

            def sub_body(s, c2):
                cv = stage_v[pl.ds(s * jnp.int32(16), 16)]
                e = jnp.full((16,), pstart + s * jnp.int32(16), jnp.int32) + iota
                ok = (e >= jnp.full((16,), start, jnp.int32)) & (
                    e < jnp.full((16,), end, jnp.int32)
                )
                col = cv & c_lo16
                val = decode_val(lax.shift_right_logical(cv, full(16)))
                rl = jnp.zeros((16,), jnp.int32)
                for oj_s in ojs:
                    rl = rl + jnp.where(
                        e >= jnp.full((16,), oj_s, jnp.int32), one, zero
                    )
                rl = jnp.where(ok, rl, c_trash)
                idx = rl * c_n + col
                b_vec = idx & full(-16)
                lane_vec = idx & full(15)
                for j in range(16):
                    b_j = pl.multiple_of(b_vec[j], 16)
                    if accumulate:
                        onehot = jnp.where(
                            iota == jnp.full((16,), lane_vec[j], jnp.int32),
                            jnp.full((16,), val[j], jnp.float32),
                            z16f,
                        )
                        plsc.addupdate(buf_v.at[pl.ds(b_j, 16)], onehot)
                    else:
                        buf_v[pl.ds(b_j, 16)] = z16f
                return c2

            lax.fori_loop(jnp.int32(0), nsub, sub_body, 0)
            return carry

        lax.fori_loop(jnp.int32(0), n_pieces, piece_body, 0)

    for g in range(n_groups):
        ovec0 = offs_v[pl.ds(g * 16, 16)]
        ovec1 = offs_v[pl.ds(g * 16 + 16, 16)]
        start = ovec0[0]
        end = ovec1[0]
        ojs = [ovec0[j] for j in range(1, 16)] + [end]
        entry_sweep(start, end, ojs, True)
        row0 = (base_row + jnp.int32(g * 16)) * jnp.int32(n)
        pltpu.sync_copy(
            buf_v.at[pl.ds(0, 16 * n)],
            dw_hbm.at[pl.ds(pl.multiple_of(row0, 8), 16 * n)],
        )
        if g + 1 < n_groups:
            entry_sweep(start, end, ojs, False)


def _outlier_dw(row_offsets, col_vals, M, N):
    offp = jnp.pad(row_offsets, (0, 144), mode="edge")
    cvp = jnp.pad(col_vals, (0, 2 * _CAP))
    mesh = plsc.VectorSubcoreMesh(core_axis_name="c", subcore_axis_name="s")
    fn = functools.partial(
        pl.kernel,
        out_type=jax.ShapeDtypeStruct((M * N,), jnp.float32),
        mesh=mesh,
        scratch_types=[
            pltpu.VMEM((144,), jnp.int32),
            pltpu.VMEM((17 * N,), jnp.float32),
            pltpu.VMEM((_CAP,), jnp.int32),
        ],
    )(_sc_scatter_body)
    return fn(offp, cvp).reshape(M, N)


def kernel(x, dense_weights, row_offsets, col_vals):
    T, N = x.shape
    M = row_offsets.shape[0] - 1
    TM, TN = M // 16, N // 16
    NT = TM * TN

    d32 = lax.bitcast_convert_type(dense_weights, jnp.int32)
    lo = d32[:, 0]
    hi = d32[:, 1]

    lo_r = lo.reshape(NT, 16).T
    hi_r = hi.reshape(NT, 16).T
    bstat = 4096
    scale_r, zero_r = pl.pallas_call(
        _stats_kernel,
        grid=(NT // bstat,),
        in_specs=[pl.BlockSpec((16, bstat), lambda i: (jnp.int32(0), i))] * 2,
        out_specs=[pl.BlockSpec((16, bstat), lambda i: (jnp.int32(0), i))] * 2,
        out_shape=[jax.ShapeDtypeStruct((16, NT), jnp.float32)] * 2,
    )(lo_r, hi_r)

    scale_T = scale_r.reshape(16, TM, TN).transpose(2, 1, 0).reshape(TN, M)
    zero_T = zero_r.reshape(16, TM, TN).transpose(2, 1, 0).reshape(TN, M)
    loT = lo.reshape(TM, TN, 16).transpose(1, 0, 2).reshape(TN, M)
    hiT = hi.reshape(TM, TN, 16).transpose(1, 0, 2).reshape(TN, M)

    dw = _outlier_dw(row_offsets, col_vals, M, N)
    dwt = (
        dw.T.reshape(N // 512, 32, 16, M)
        .transpose(0, 2, 1, 3)
        .reshape(N, M)
    )

    x_bf = (
        x.reshape(T, N // 512, 32, 16)
        .transpose(0, 1, 3, 2)
        .reshape(T, N)
        .astype(jnp.bfloat16)
    )

    bm = 512
    bk = 512
    btn = bk // 16
    grid = (M // bm, N // bk)
    out = pl.pallas_call(
        _mm_kernel,
        grid=grid,
        in_specs=[
            pl.BlockSpec((T, bk), lambda m, k: (jnp.int32(0), k)),
            pl.BlockSpec((btn, bm), lambda m, k: (k, m)),
            pl.BlockSpec((btn, bm), lambda m, k: (k, m)),
            pl.BlockSpec((btn, bm), lambda m, k: (k, m)),
            pl.BlockSpec((btn, bm), lambda m, k: (k, m)),
            pl.BlockSpec((bk, bm), lambda m, k: (k, m)),
        ],
        out_specs=pl.BlockSpec((T, bm), lambda m, k: (jnp.int32(0), m)),
        out_shape=jax.ShapeDtypeStruct((T, M), jnp.float32),
        compiler_params=pltpu.CompilerParams(
            dimension_semantics=("parallel", "arbitrary"),
        ),
    )(x_bf, loT, hiT, scale_T, zero_T, dwt)
    return out


# speedup vs baseline: 1.2873x; 1.2873x over previous
import functools

import jax
import jax.numpy as jnp
from jax import lax
from jax.experimental import pallas as pl
from jax.experimental.pallas import tpu as pltpu
from jax.experimental.pallas import tpu_sc as plsc

_NSC = 2
_NSUB = 16
_NW = _NSC * _NSUB
_CAP = 4096


def _srl(x, n):
    return lax.shift_right_logical(x, jnp.int32(n))


def _sll(x, n):
    return lax.shift_left(x, jnp.int32(n))


def _fp16_bits_to_f32(u):
    s = _srl(u, 15) & 1
    e = _srl(u, 10) & 31
    man = u & 1023
    bits = _sll(s, 31) | _sll(e + 112, 23) | _sll(man, 13)
    val_n = lax.bitcast_convert_type(bits, jnp.float32)
    sgn = 1.0 - 2.0 * s.astype(jnp.float32)
    val_s = sgn * man.astype(jnp.float32) * jnp.float32(2.0 ** -24)
    return jnp.where(e == 0, val_s, val_n)


def _stats_kernel(lo_ref, hi_ref, scale_ref, zero_ref):
    lo = lo_ref[...]
    hi = hi_ref[...]
    c0 = _srl(hi, 22) & 255

    def so(j):
        u = c0[2 * j : 2 * j + 1, :] | _sll(c0[2 * j + 1 : 2 * j + 2, :], 8)
        return _fp16_bits_to_f32(u)

    ss, sz, zs, zz = so(0), so(1), so(2), so(3)
    ws = (lo & 7).astype(jnp.float32)
    wz = (_srl(lo, 3) & 7).astype(jnp.float32)
    scale_ref[...] = ws * ss + sz
    zero_ref[...] = wz * zs + zz


def _mm_kernel(x_ref, lo_ref, hi_ref, sc_ref, zr_ref, dwt_ref, out_ref):
    lo = lo_ref[...]
    hi = hi_ref[...]
    planes = []
    for i in range(16):
        s = 6 + 3 * i
        if s + 3 <= 32:
            p = _srl(lo, s) & 7
        elif s < 32:
            p = (_srl(lo, 30) & 3) | _sll(hi & 1, 2)
        else:
            p = _srl(hi, s - 32) & 7
        planes.append(p)
    wq = jnp.stack(planes, axis=1).astype(jnp.float32)
    sc = sc_ref[...]
    zr = zr_ref[...]
    w = sc[:, None, :] * (wq - zr[:, None, :])
    btn, _, bm = w.shape
    w = w.reshape(btn * 16, bm) + dwt_ref[...].T
    acc = jnp.dot(
        x_ref[...], w.astype(jnp.bfloat16), preferred_element_type=jnp.float32
    )
    k = pl.program_id(1)

    @pl.when(k == 0)
    def _():
        out_ref[...] = acc

    @pl.when(k > 0)
    def _():
        out_ref[...] += acc


def _sc_scatter_body(offs_hbm, cv_hbm, dw_hbm, offs_v, buf_v, stage_v):
    mn = dw_hbm.shape[0]
    n = buf_v.shape[0] // 17
    m = mn // n
    rows_per_w = m // _NW
    n_groups = rows_per_w // 16
    wid = lax.axis_index("s") * jnp.int32(_NSC) + lax.axis_index("c")
    base_row = wid * jnp.int32(rows_per_w)
    pltpu.sync_copy(offs_hbm.at[pl.ds(pl.multiple_of(base_row, 8), 144)], offs_v)

    def full(v):
        return jnp.full((16,), v, jnp.int32)

    z16f = jnp.zeros((16,), jnp.float32)
    iota = lax.iota(jnp.int32, 16)
    c_n = full(n)
    c_trash = full(16)
    c_lo16 = full(0xFFFF)
    one = full(1)
    zero = full(0)

    def decode_val(u):
        s = lax.shift_right_logical(u, full(15)) & one
        ex = lax.shift_right_logical(u, full(10)) & full(31)
        man = u & full(1023)
        bits = (
            lax.shift_left(s, full(31))
            | lax.shift_left(ex + full(112), full(23))
            | lax.shift_left(man, full(13))
        )
        val_n = lax.bitcast_convert_type(bits, jnp.float32)
        sgn = jnp.full((16,), 1.0, jnp.float32) - jnp.full(
            (16,), 2.0, jnp.float32
        ) * s.astype(jnp.float32)
        val_s = (
            sgn
            * man.astype(jnp.float32)
            * jnp.full((16,), 2.0 ** -24, jnp.float32)
        )
        return jnp.where(ex == zero, val_s, val_n)

    def zero_body(i, carry):
        buf_v[pl.ds(i * jnp.int32(16), 16)] = z16f
        return carry

    lax.fori_loop(jnp.int32(0), jnp.int32(17 * n // 16), zero_body, 0)

    def entry_sweep(start, end, ojs, accumulate):
        p0 = start & jnp.int32(-8)
        n_pieces = lax.div(end - p0 + jnp.int32(_CAP - 1), jnp.int32(_CAP))

        def piece_body(pi, carry):
            pstart = p0 + pi * jnp.int32(_CAP)
            pltpu.sync_copy(
                cv_hbm.at[pl.ds(pl.multiple_of(pstart, 8), _CAP)], stage_v
            )
            nsub = lax.div(
                jnp.minimum(end - pstart, jnp.int32(_CAP)) + jnp.int32(15),
                jnp.int32(16),
            )

            def sub_body(s, c2):
                cv = stage_v[pl.ds(s * jnp.int32(16), 16)]
                e = jnp.full((16,), pstart + s * jnp.int32(16), jnp.int32) + iota
                ok = (e >= jnp.full((16,), start, jnp.int32)) & (
                    e < jnp.full((16,), end, jnp.int32)
                )
                col = cv & c_lo16
                val = decode_val(lax.shift_right_logical(cv, full(16)))
                rl = jnp.zeros((16,), jnp.int32)
                for oj_s in ojs:
                    rl = rl + jnp.where(
                        e >= jnp.full((16,), oj_s, jnp.int32), one, zero
                    )
                rl = jnp.where(ok, rl, c_trash)
                idx = rl * c_n + col
                b_vec = idx & full(-16)
                lane_vec = idx & full(15)
                for j in range(16):
                    b_j = pl.multiple_of(b_vec[j], 16)
                    if accumulate:
                        onehot = jnp.where(
                            iota == jnp.full((16,), lane_vec[j], jnp.int32),
                            jnp.full((16,), val[j], jnp.float32),
                            z16f,
                        )
                        plsc.addupdate(buf_v.at[pl.ds(b_j, 16)], onehot)
                    else:
                        buf_v[pl.ds(b_j, 16)] = z16f
                return c2

            lax.fori_loop(jnp.int32(0), nsub, sub_body, 0)
            return carry

        lax.fori_loop(jnp.int32(0), n_pieces, piece_body, 0)

    for g in range(n_groups):
        ovec0 = offs_v[pl.ds(g * 16, 16)]
        ovec1 = offs_v[pl.ds(g * 16 + 16, 16)]
        start = ovec0[0]
        end = ovec1[0]
        ojs = [ovec0[j] for j in range(1, 16)] + [end]
        entry_sweep(start, end, ojs, True)
        row0 = (base_row + jnp.int32(g * 16)) * jnp.int32(n)
        pltpu.sync_copy(
            buf_v.at[pl.ds(0, 16 * n)],
            dw_hbm.at[pl.ds(pl.multiple_of(row0, 8), 16 * n)],
        )
        if g + 1 < n_groups:
            entry_sweep(start, end, ojs, False)


def _outlier_dw(row_offsets, col_vals, M, N):
    offp = jnp.pad(row_offsets, (0, 144), mode="edge")
    cvp = jnp.pad(col_vals, (0, 2 * _CAP))
    mesh = plsc.VectorSubcoreMesh(core_axis_name="c", subcore_axis_name="s")
    fn = functools.partial(
        pl.kernel,
        out_type=jax.ShapeDtypeStruct((M * N,), jnp.float32),
        mesh=mesh,
        scratch_types=[
            pltpu.VMEM((144,), jnp.int32),
            pltpu.VMEM((17 * N,), jnp.float32),
            pltpu.VMEM((_CAP,), jnp.int32),
        ],
    )(_sc_scatter_body)
    return fn(offp, cvp).reshape(M, N)


def kernel(x, dense_weights, row_offsets, col_vals):
    T, N = x.shape
    M = row_offsets.shape[0] - 1
    TM, TN = M // 16, N // 16
    NT = TM * TN

    d32 = lax.bitcast_convert_type(dense_weights, jnp.int32)
    lo = d32[:, 0]
    hi = d32[:, 1]

    lo_r = lo.reshape(NT, 16).T
    hi_r = hi.reshape(NT, 16).T
    bstat = 4096
    scale_r, zero_r = pl.pallas_call(
        _stats_kernel,
        grid=(NT // bstat,),
        in_specs=[pl.BlockSpec((16, bstat), lambda i: (jnp.int32(0), i))] * 2,
        out_specs=[pl.BlockSpec((16, bstat), lambda i: (jnp.int32(0), i))] * 2,
        out_shape=[jax.ShapeDtypeStruct((16, NT), jnp.float32)] * 2,
    )(lo_r, hi_r)

    scale_T = scale_r.reshape(16, TM, TN).transpose(2, 1, 0).reshape(TN, M)
    zero_T = zero_r.reshape(16, TM, TN).transpose(2, 1, 0).reshape(TN, M)
    loT = lo.reshape(TM, TN, 16).transpose(1, 0, 2).reshape(TN, M)
    hiT = hi.reshape(TM, TN, 16).transpose(1, 0, 2).reshape(TN, M)

    dw = _outlier_dw(row_offsets, col_vals, M, N)

    x_bf = x.astype(jnp.bfloat16)

    bm = 512
    bk = 512
    btn = bk // 16
    grid = (M // bm, N // bk)
    out = pl.pallas_call(
        _mm_kernel,
        grid=grid,
        in_specs=[
            pl.BlockSpec((T, bk), lambda m, k: (jnp.int32(0), k)),
            pl.BlockSpec((btn, bm), lambda m, k: (k, m)),
            pl.BlockSpec((btn, bm), lambda m, k: (k, m)),
            pl.BlockSpec((btn, bm), lambda m, k: (k, m)),
            pl.BlockSpec((btn, bm), lambda m, k: (k, m)),
            pl.BlockSpec((bm, bk), lambda m, k: (m, k)),
        ],
        out_specs=pl.BlockSpec((T, bm), lambda m, k: (jnp.int32(0), m)),
        out_shape=jax.ShapeDtypeStruct((T, M), jnp.float32),
        compiler_params=pltpu.CompilerParams(
            dimension_semantics=("parallel", "arbitrary"),
        ),
    )(x_bf, loT, hiT, scale_T, zero_T, dw)
    return out


# speedup vs baseline: 1.3330x; 1.0355x over previous
import functools

import jax
import jax.numpy as jnp
from jax import lax
from jax.experimental import pallas as pl
from jax.experimental.pallas import tpu as pltpu
from jax.experimental.pallas import tpu_sc as plsc

_NSC = 2
_NSUB = 16
_NW = _NSC * _NSUB
_CAP = 4096


def _srl(x, n):
    return lax.shift_right_logical(x, jnp.int32(n))


def _sll(x, n):
    return lax.shift_left(x, jnp.int32(n))


def _fp16_bits_to_f32(u):
    s = _srl(u, 15) & 1
    e = _srl(u, 10) & 31
    man = u & 1023
    bits = _sll(s, 31) | _sll(e + 112, 23) | _sll(man, 13)
    val_n = lax.bitcast_convert_type(bits, jnp.float32)
    sgn = 1.0 - 2.0 * s.astype(jnp.float32)
    val_s = sgn * man.astype(jnp.float32) * jnp.float32(2.0 ** -24)
    return jnp.where(e == 0, val_s, val_n)


def _stats_kernel(lo_ref, hi_ref, scale_ref, zero_ref):
    lo = lo_ref[...]
    hi = hi_ref[...]
    c0 = _srl(hi, 22) & 255

    def so(j):
        u = c0[2 * j : 2 * j + 1, :] | _sll(c0[2 * j + 1 : 2 * j + 2, :], 8)
        return _fp16_bits_to_f32(u)

    ss, sz, zs, zz = so(0), so(1), so(2), so(3)
    ws = (lo & 7).astype(jnp.float32)
    wz = (_srl(lo, 3) & 7).astype(jnp.float32)
    scale_ref[...] = ws * ss + sz
    zero_ref[...] = wz * zs + zz


def _mm_kernel(x_ref, lo_ref, hi_ref, sc_ref, zr_ref, dwt_ref, out_ref):
    lo = lo_ref[...]
    hi = hi_ref[...]
    planes = []
    for i in range(16):
        s = 6 + 3 * i
        if s + 3 <= 32:
            p = _srl(lo, s) & 7
        elif s < 32:
            p = (_srl(lo, 30) & 3) | _sll(hi & 1, 2)
        else:
            p = _srl(hi, s - 32) & 7
        planes.append(p)
    wq = jnp.stack(planes, axis=1).astype(jnp.float32)
    sc = sc_ref[...]
    zr = zr_ref[...]
    w = sc[:, None, :] * (wq - zr[:, None, :])
    btn, _, bm = w.shape
    w = w.reshape(btn * 16, bm) + dwt_ref[...].T
    acc = jnp.dot(
        x_ref[...], w.astype(jnp.bfloat16), preferred_element_type=jnp.float32
    )
    k = pl.program_id(1)

    @pl.when(k == 0)
    def _():
        out_ref[...] = acc

    @pl.when(k > 0)
    def _():
        out_ref[...] += acc


def _sc_scatter_body(offs_hbm, cv_hbm, dw_hbm, offs_v, buf_v, stage_v):
    mn = dw_hbm.shape[0]
    n = buf_v.shape[0] // 17
    m = mn // n
    rows_per_w = m // _NW
    n_groups = rows_per_w // 16
    wid = lax.axis_index("s") * jnp.int32(_NSC) + lax.axis_index("c")
    base_row = wid * jnp.int32(rows_per_w)
    pltpu.sync_copy(offs_hbm.at[pl.ds(pl.multiple_of(base_row, 8), 144)], offs_v)

    def full(v):
        return jnp.full((16,), v, jnp.int32)

    z16f = jnp.zeros((16,), jnp.float32)
    iota = lax.iota(jnp.int32, 16)
    c_n = full(n)
    c_trash = full(16)
    c_lo16 = full(0xFFFF)
    one = full(1)
    zero = full(0)

    def decode_val(u):
        s = lax.shift_right_logical(u, full(15)) & one
        ex = lax.shift_right_logical(u, full(10)) & full(31)
        man = u & full(1023)
        bits = (
            lax.shift_left(s, full(31))
            | lax.shift_left(ex + full(112), full(23))
            | lax.shift_left(man, full(13))
        )
        val_n = lax.bitcast_convert_type(bits, jnp.float32)
        sgn = jnp.full((16,), 1.0, jnp.float32) - jnp.full(
            (16,), 2.0, jnp.float32
        ) * s.astype(jnp.float32)
        val_s = (
            sgn
            * man.astype(jnp.float32)
            * jnp.full((16,), 2.0 ** -24, jnp.float32)
        )
        return jnp.where(ex == zero, val_s, val_n)

    def zero_body(i, carry):
        buf_v[pl.ds(i * jnp.int32(16), 16)] = z16f
        return carry

    lax.fori_loop(jnp.int32(0), jnp.int32(17 * n // 16), zero_body, 0)

    def entry_sweep(start, end, ojs, accumulate):
        p0 = start & jnp.int32(-8)
        n_pieces = lax.div(end - p0 + jnp.int32(_CAP - 1), jnp.int32(_CAP))

        def piece_body(pi, carry):
            pstart = p0 + pi * jnp.int32(_CAP)
            pltpu.sync_copy(
                cv_hbm.at[pl.ds(pl.multiple_of(pstart, 8), _CAP)], stage_v
            )
            nsub = lax.div(
                jnp.minimum(end - pstart, jnp.int32(_CAP)) + jnp.int32(15),
                jnp.int32(16),
            )

            def sub_body(s, c2):
                cv = stage_v[pl.ds(s * jnp.int32(16), 16)]
                e = jnp.full((16,), pstart + s * jnp.int32(16), jnp.int32) + iota
                ok = (e >= jnp.full((16,), start, jnp.int32)) & (
                    e < jnp.full((16,), end, jnp.int32)
                )
                col = cv & c_lo16
                val = decode_val(lax.shift_right_logical(cv, full(16)))
                rl = jnp.zeros((16,), jnp.int32)
                for oj_s in ojs:
                    rl = rl + jnp.where(
                        e >= jnp.full((16,), oj_s, jnp.int32), one, zero
                    )
                rl = jnp.where(ok, rl, c_trash)
                idx = rl * c_n + col
                b_vec = idx & full(-16)
                lane_vec = idx & full(15)
                for j in range(16):
                    b_j = pl.multiple_of(b_vec[j], 16)
                    if accumulate:
                        onehot = jnp.where(
                            iota == jnp.full((16,), lane_vec[j], jnp.int32),
                            jnp.full((16,), val[j], jnp.float32),
                            z16f,
                        )
                        plsc.addupdate(buf_v.at[pl.ds(b_j, 16)], onehot)
                    else:
                        buf_v[pl.ds(b_j, 16)] = z16f
                return c2

            lax.fori_loop(jnp.int32(0), nsub, sub_body, 0)
            return carry

        lax.fori_loop(jnp.int32(0), n_pieces, piece_body, 0)

    for g in range(n_groups):
        ovec0 = offs_v[pl.ds(g * 16, 16)]
        ovec1 = offs_v[pl.ds(g * 16 + 16, 16)]
        start = ovec0[0]
        end = ovec1[0]
        ojs = [ovec0[j] for j in range(1, 16)] + [end]
        entry_sweep(start, end, ojs, True)
        row0 = (base_row + jnp.int32(g * 16)) * jnp.int32(n)
        pltpu.sync_copy(
            buf_v.at[pl.ds(0, 16 * n)],
            dw_hbm.at[pl.ds(pl.multiple_of(row0, 8), 16 * n)],
        )
        if g + 1 < n_groups:
            entry_sweep(start, end, ojs, False)


def _outlier_dw(row_offsets, col_vals, M, N):
    offp = jnp.pad(row_offsets, (0, 144), mode="edge")
    cvp = jnp.pad(col_vals, (0, 2 * _CAP))
    mesh = plsc.VectorSubcoreMesh(core_axis_name="c", subcore_axis_name="s")
    fn = functools.partial(
        pl.kernel,
        out_type=jax.ShapeDtypeStruct((M * N,), jnp.float32),
        mesh=mesh,
        scratch_types=[
            pltpu.VMEM((144,), jnp.int32),
            pltpu.VMEM((17 * N,), jnp.float32),
            pltpu.VMEM((_CAP,), jnp.int32),
        ],
    )(_sc_scatter_body)
    return fn(offp, cvp).reshape(M, N)


def kernel(x, dense_weights, row_offsets, col_vals):
    T, N = x.shape
    M = row_offsets.shape[0] - 1
    TM, TN = M // 16, N // 16
    NT = TM * TN

    d32 = lax.bitcast_convert_type(dense_weights, jnp.int32)
    lo = d32[:, 0]
    hi = d32[:, 1]

    lo_r = lo.reshape(NT, 16).T
    hi_r = hi.reshape(NT, 16).T
    bstat = 4096
    scale_r, zero_r = pl.pallas_call(
        _stats_kernel,
        grid=(NT // bstat,),
        in_specs=[pl.BlockSpec((16, bstat), lambda i: (jnp.int32(0), i))] * 2,
        out_specs=[pl.BlockSpec((16, bstat), lambda i: (jnp.int32(0), i))] * 2,
        out_shape=[jax.ShapeDtypeStruct((16, NT), jnp.float32)] * 2,
    )(lo_r, hi_r)

    scale_T = scale_r.reshape(16, TM, TN).transpose(2, 1, 0).reshape(TN, M)
    zero_T = zero_r.reshape(16, TM, TN).transpose(2, 1, 0).reshape(TN, M)
    loT = lo.reshape(TM, TN, 16).transpose(1, 0, 2).reshape(TN, M)
    hiT = hi.reshape(TM, TN, 16).transpose(1, 0, 2).reshape(TN, M)

    dw = _outlier_dw(row_offsets, col_vals, M, N)

    x_bf = x.astype(jnp.bfloat16)

    bm = 512
    bk = 1024
    btn = bk // 16
    grid = (M // bm, N // bk)
    out = pl.pallas_call(
        _mm_kernel,
        grid=grid,
        in_specs=[
            pl.BlockSpec((T, bk), lambda m, k: (jnp.int32(0), k)),
            pl.BlockSpec((btn, bm), lambda m, k: (k, m)),
            pl.BlockSpec((btn, bm), lambda m, k: (k, m)),
            pl.BlockSpec((btn, bm), lambda m, k: (k, m)),
            pl.BlockSpec((btn, bm), lambda m, k: (k, m)),
            pl.BlockSpec((bm, bk), lambda m, k: (m, k)),
        ],
        out_specs=pl.BlockSpec((T, bm), lambda m, k: (jnp.int32(0), m)),
        out_shape=jax.ShapeDtypeStruct((T, M), jnp.float32),
        compiler_params=pltpu.CompilerParams(
            dimension_semantics=("parallel", "arbitrary"),
        ),
    )(x_bf, loT, hiT, scale_T, zero_T, dw)
    return out
